# baseline (speedup 1.0000x reference)
import jax
import jax.numpy as jnp
from jax.experimental import pallas as pl
from jax.experimental.pallas import tpu as pltpu

B, N, H, W, D, GE, EPG = 32, 512, 64, 64, 128, 64, 8192
P = H * W
EC = 4096
PC = 4096

_NT = (((1,), (1,)), ((), ()))
_TN = (((0,), (0,)), ((), ()))


def _graph_kernel(x_ref, src_ref, dst_ref, mp_ref, w1_ref, w2_ref, w3_ref,
                  rep_ref, ag_ref):
    iota_e = jax.lax.broadcasted_iota(jnp.int16, (N, EC), 0)
    f8 = jnp.float8_e4m3fn
    bone = jnp.ones((), jnp.bfloat16)
    bzero = jnp.zeros((), jnp.bfloat16)
    a = jnp.zeros((N, N), jnp.float32)
    for c in range(EPG // EC):
        sv = src_ref[0, :, c * EC:(c + 1) * EC].astype(jnp.int16)
        dv = dst_ref[0, :, c * EC:(c + 1) * EC].astype(jnp.int16)
        ohs = jnp.where(iota_e == sv, bone, bzero).astype(f8)
        ohd = jnp.where(iota_e == dv, bone, bzero).astype(f8)
        a = a + jax.lax.dot_general(ohd, ohs, _NT,
                                    preferred_element_type=jnp.float32)
    deg_in = jnp.maximum(jnp.sum(a, axis=1, keepdims=True), 1.0)
    deg_out = jnp.maximum(jnp.sum(a, axis=0, keepdims=True), 1.0)
    an = a * jax.lax.rsqrt(deg_in) * jax.lax.rsqrt(deg_out)

    anb = an.astype(jnp.bfloat16)

    def gconv(v, w):
        t = jnp.dot(v.astype(jnp.bfloat16), w.astype(jnp.bfloat16),
                    preferred_element_type=jnp.float32).astype(jnp.bfloat16)
        return jnp.dot(anb, t, preferred_element_type=jnp.float32)

    x = x_ref[0]
    h = jnp.maximum(gconv(x, w1_ref[...]), 0.0)
    h = jnp.maximum(gconv(h, w2_ref[...]), 0.0)
    g = jnp.tanh(gconv(h, w3_ref[...]))

    iota_p = jax.lax.broadcasted_iota(jnp.int16, (N, PC), 0)
    gb = g.astype(jnp.bfloat16)
    for c in range(P // PC):
        mpv = mp_ref[0, :, c * PC:(c + 1) * PC].astype(jnp.int16)
        ohm = jnp.where(iota_p == mpv, bone, bzero)
        rep_ref[0, c * PC:(c + 1) * PC, :] = jax.lax.dot_general(
            ohm, gb, _TN, preferred_element_type=jnp.float32).astype(jnp.bfloat16)
    ag_ref[0] = g[0:1, :]


def _shift2(x3, di, dj):
    h_, w_, c_ = x3.shape
    if di > 0:
        x3 = jnp.concatenate([x3[di:], jnp.zeros((di, w_, c_), x3.dtype)], 0)
    elif di < 0:
        x3 = jnp.concatenate([jnp.zeros((-di, w_, c_), x3.dtype), x3[:di]], 0)
    if dj > 0:
        x3 = jnp.concatenate([x3[:, dj:], jnp.zeros((h_, dj, c_), x3.dtype)], 1)
    elif dj < 0:
        x3 = jnp.concatenate([jnp.zeros((h_, -dj, c_), x3.dtype), x3[:, :dj]], 1)
    return x3


def _conv3x3(x3, w_ref, b_ref, cout, offs=(-1, 0, 1)):
    s, _, c = x3.shape
    xb = x3.astype(jnp.bfloat16)
    xcat = jnp.concatenate([_shift2(xb, offs[0], 0), _shift2(xb, offs[1], 0),
                            _shift2(xb, offs[2], 0)], 2)
    yw = jnp.dot(xcat.reshape(s * s, 3 * c), w_ref[...],
                 preferred_element_type=jnp.float32)
    y3 = yw.reshape(s, s, 3 * cout)
    acc = b_ref[...].reshape(1, 1, cout)
    for kj in range(3):
        acc = acc + _shift2(y3[:, :, kj * cout:(kj + 1) * cout], 0, offs[kj])
    return acc.reshape(s * s, cout)


def _mask_sq(y, s, c, valid):
    ii = jax.lax.broadcasted_iota(jnp.int32, (s, s, 1), 0)
    jj = jax.lax.broadcasted_iota(jnp.int32, (s, s, 1), 1)
    return jnp.where((ii < valid) & (jj < valid), y.reshape(s, s, c), 0.0)


def _conv4x4s2(xref, w_ref, b_ref):
    s, _, cin = xref.shape
    half = s // 2
    par = [[xref[slice(p, s, 2), slice(q, s, 2), :] for q in range(2)]
           for p in range(2)]
    acc = b_ref[...]
    for ki in range(4):
        a_, p_ = (ki - 1) >> 1, (ki - 1) & 1
        for kj in range(4):
            b_, q_ = (kj - 1) >> 1, (kj - 1) & 1
            xs = _shift2(par[p_][q_], a_, b_).reshape(half * half, cin)
            acc = acc + jnp.dot(xs.astype(jnp.bfloat16), w_ref[ki * 4 + kj],
                                preferred_element_type=jnp.float32)
    return acc


def _head_kernel(rep1_ref, rep2_ref, ag1_ref, ag2_ref,
                 c1w_ref, c1b_ref, c2w_ref, c2b_ref, c3w_ref, c3b_ref,
                 d1w_ref, d1b_ref, d2w_ref, d2b_ref, d3w_ref, d3b_ref,
                 d4w_ref, d4b_ref, a1w_ref, a1b_ref, a2w_ref, a2b_ref,
                 res_ref, rm2_ref, s1_ref, s2_ref, s3_ref):
    def conv_stack(rep):
        y = jnp.maximum(_conv3x3(rep.reshape(H, W, GE), c1w_ref, c1b_ref, 256), 0.0)
        y = jnp.maximum(_conv3x3(y.reshape(H, W, 256), c2w_ref, c2b_ref, 128), 0.0)
        return jnp.tanh(_conv3x3(y.reshape(H, W, 128), c3w_ref, c3b_ref, GE))

    rm1 = conv_stack(rep1_ref[0])
    rm2 = conv_stack(rep2_ref[0])
    rm2_ref[0] = rm2

    xa = jnp.concatenate([rm1[0:1, :], ag1_ref[0], rm2[0:1, :], ag2_ref[0]],
                         axis=1)
    ha = jnp.maximum(jnp.dot(xa, a1w_ref[...],
                             preferred_element_type=jnp.float32) + a1b_ref[...],
                     0.0)
    ares = jnp.dot(ha, a2w_ref[...],
                   preferred_element_type=jnp.float32) + a2b_ref[...]

    xm = jnp.concatenate([rm1, rm2], axis=1).reshape(H, W, 2 * GE)
    y = jnp.maximum(_conv3x3(xm, d1w_ref, d1b_ref, 32, offs=(0, 1, 2)), 0.0)
    s1_ref[...] = _mask_sq(y, H, 32, 62)
    y = jnp.maximum(_conv4x4s2(s1_ref, d2w_ref, d2b_ref), 0.0)
    s2_ref[...] = _mask_sq(y, 32, 16, 31)
    y = jnp.maximum(_conv4x4s2(s2_ref, d3w_ref, d3b_ref), 0.0)
    s3_ref[...] = _mask_sq(y, 16, 8, 15)
    y = _conv4x4s2(s3_ref, d4w_ref, d4b_ref)
    y88 = y.reshape(8, 8, 1)[:, :, 0]
    res = (jax.nn.sigmoid(y88) + jax.nn.sigmoid(ares)) / 2.0
    res_ref[0, 0] = res[0:7, 0:7]


def _full(shape):
    return pl.BlockSpec(shape, lambda i: (0,) * len(shape))


def kernel(x1, src1, dst1, mp1, x2, src2, dst2, mp2,
           W1, W2, W3, c1w, c1b, c2w, c2b, c3w, c3b,
           d1w, d1b, d2w, d2b, d3w, d3b, d4w, d4b,
           a1w, a1b, a2w, a2b):
    f32 = jnp.float32
    offs = (jnp.arange(B, dtype=jnp.int32) * N)[:, None]
    xs = jnp.concatenate([x1.reshape(B, N, D), x2.reshape(B, N, D)], 0)
    srcs = jnp.concatenate([(src1.reshape(B, EPG) - offs),
                            (src2.reshape(B, EPG) - offs)], 0).reshape(2 * B, 1, EPG)
    dsts = jnp.concatenate([(dst1.reshape(B, EPG) - offs),
                            (dst2.reshape(B, EPG) - offs)], 0).reshape(2 * B, 1, EPG)
    mps = jnp.concatenate([mp1.reshape(B, 1, P), mp2.reshape(B, 1, P)], 0)

    rep, ag = pl.pallas_call(
        _graph_kernel,
        grid=(2 * B,),
        in_specs=[
            pl.BlockSpec((1, N, D), lambda i: (i, 0, 0)),
            pl.BlockSpec((1, 1, EPG), lambda i: (i, 0, 0)),
            pl.BlockSpec((1, 1, EPG), lambda i: (i, 0, 0)),
            pl.BlockSpec((1, 1, P), lambda i: (i, 0, 0)),
            pl.BlockSpec((D, D), lambda i: (0, 0)),
            pl.BlockSpec((D, D), lambda i: (0, 0)),
            pl.BlockSpec((D, GE), lambda i: (0, 0)),
        ],
        out_specs=[
            pl.BlockSpec((1, P, GE), lambda i: (i, 0, 0)),
            pl.BlockSpec((1, 1, GE), lambda i: (i, 0, 0)),
        ],
        out_shape=[
            jax.ShapeDtypeStruct((2 * B, P, GE), jnp.bfloat16),
            jax.ShapeDtypeStruct((2 * B, 1, GE), f32),
        ],
        compiler_params=pltpu.CompilerParams(
            dimension_semantics=("parallel",),
            vmem_limit_bytes=48 * 1024 * 1024,
        ),
        name="graph_conv_map",
    )(xs, srcs, dsts, mps, W1, W2, W3)

    def taps(w):
        return jnp.transpose(w, (2, 3, 1, 0)).reshape(-1, w.shape[1], w.shape[0])

    def btaps(w):
        return taps(w).astype(jnp.bfloat16)

    def vtaps(w):
        t = jnp.transpose(w, (2, 1, 3, 0))
        return t.reshape(3 * w.shape[1], 3 * w.shape[0]).astype(jnp.bfloat16)

    res, rm2 = pl.pallas_call(
        _head_kernel,
        grid=(B,),
        in_specs=[
            pl.BlockSpec((1, P, GE), lambda i: (i, 0, 0)),
            pl.BlockSpec((1, P, GE), lambda i: (i, 0, 0)),
            pl.BlockSpec((1, 1, GE), lambda i: (i, 0, 0)),
            pl.BlockSpec((1, 1, GE), lambda i: (i, 0, 0)),
            _full((3 * GE, 3 * 256)), _full((1, 256)),
            _full((3 * 256, 3 * 128)), _full((1, 128)),
            _full((3 * 128, 3 * GE)), _full((1, GE)),
            _full((3 * 128, 3 * 32)), _full((1, 32)),
            _full((16, 32, 16)), _full((1, 16)),
            _full((16, 16, 8)), _full((1, 8)),
            _full((16, 8, 1)), _full((1, 1)),
            _full((2 * (64 + GE), 32)), _full((1, 32)),
            _full((32, 1)), _full((1, 1)),
        ],
        out_specs=[
            pl.BlockSpec((1, 1, 7, 7), lambda i: (i, 0, 0, 0)),
            pl.BlockSpec((1, P, GE), lambda i: (i, 0, 0)),
        ],
        out_shape=[
            jax.ShapeDtypeStruct((B, 1, 7, 7), f32),
            jax.ShapeDtypeStruct((B, P, GE), f32),
        ],
        scratch_shapes=[
            pltpu.VMEM((H, W, 32), jnp.float32),
            pltpu.VMEM((32, 32, 16), jnp.float32),
            pltpu.VMEM((16, 16, 8), jnp.float32),
        ],
        compiler_params=pltpu.CompilerParams(
            dimension_semantics=("parallel",),
            vmem_limit_bytes=48 * 1024 * 1024,
        ),
        name="conv_head",
    )(rep[:B], rep[B:], ag[:B], ag[B:],
      vtaps(c1w), c1b[None, :], vtaps(c2w), c2b[None, :], vtaps(c3w), c3b[None, :],
      vtaps(d1w), d1b[None, :], btaps(d2w), d2b[None, :], btaps(d3w), d3b[None, :],
      btaps(d4w), d4b[None, :],
      a1w.T, a1b[None, :], a2w.T, a2b[None, :])

    rm2_nchw = jnp.transpose(rm2.reshape(B, H, W, GE), (0, 3, 1, 2))
    return res, rm2_nchw


# speedup vs baseline: 1.0085x; 1.0085x over previous
import jax
import jax.numpy as jnp
from jax.experimental import pallas as pl
from jax.experimental.pallas import tpu as pltpu

B, N, H, W, D, GE, EPG = 32, 512, 64, 64, 128, 64, 8192
P = H * W
EC = 2048
PC = 2048

_NT = (((1,), (1,)), ((), ()))
_TN = (((0,), (0,)), ((), ()))


def _graph_kernel(x_ref, src_ref, dst_ref, mp_ref, w1_ref, w2_ref, w3_ref,
                  rep_ref, ag_ref):
    iota_e = jax.lax.broadcasted_iota(jnp.int16, (N, EC), 0)
    f8 = jnp.float8_e4m3fn
    bone = jnp.ones((), jnp.bfloat16)
    bzero = jnp.zeros((), jnp.bfloat16)
    a = jnp.zeros((N, N), jnp.float32)
    for c in range(EPG // EC):
        sv = src_ref[0, :, c * EC:(c + 1) * EC].astype(jnp.int16)
        dv = dst_ref[0, :, c * EC:(c + 1) * EC].astype(jnp.int16)
        ohs = jnp.where(iota_e == sv, bone, bzero).astype(f8)
        ohd = jnp.where(iota_e == dv, bone, bzero).astype(f8)
        a = a + jax.lax.dot_general(ohd, ohs, _NT,
                                    preferred_element_type=jnp.float32)
    deg_in = jnp.maximum(jnp.sum(a, axis=1, keepdims=True), 1.0)
    deg_out = jnp.maximum(jnp.sum(a, axis=0, keepdims=True), 1.0)
    an = a * jax.lax.rsqrt(deg_in) * jax.lax.rsqrt(deg_out)

    anb = an.astype(jnp.bfloat16)

    def gconv(v, w):
        t = jnp.dot(v.astype(jnp.bfloat16), w.astype(jnp.bfloat16),
                    preferred_element_type=jnp.float32).astype(jnp.bfloat16)
        return jnp.dot(anb, t, preferred_element_type=jnp.float32)

    x = x_ref[0]
    h = jnp.maximum(gconv(x, w1_ref[...]), 0.0)
    h = jnp.maximum(gconv(h, w2_ref[...]), 0.0)
    g = jnp.tanh(gconv(h, w3_ref[...]))

    iota_p = jax.lax.broadcasted_iota(jnp.int16, (N, PC), 0)
    gb = g.astype(jnp.bfloat16)
    for c in range(P // PC):
        mpv = mp_ref[0, :, c * PC:(c + 1) * PC].astype(jnp.int16)
        ohm = jnp.where(iota_p == mpv, bone, bzero)
        rep_ref[0, c * PC:(c + 1) * PC, :] = jax.lax.dot_general(
            ohm, gb, _TN, preferred_element_type=jnp.float32).astype(jnp.bfloat16)
    ag_ref[0] = g[0:1, :]


def _shift2(x3, di, dj):
    h_, w_, c_ = x3.shape
    if di > 0:
        x3 = jnp.concatenate([x3[di:], jnp.zeros((di, w_, c_), x3.dtype)], 0)
    elif di < 0:
        x3 = jnp.concatenate([jnp.zeros((-di, w_, c_), x3.dtype), x3[:di]], 0)
    if dj > 0:
        x3 = jnp.concatenate([x3[:, dj:], jnp.zeros((h_, dj, c_), x3.dtype)], 1)
    elif dj < 0:
        x3 = jnp.concatenate([jnp.zeros((h_, -dj, c_), x3.dtype), x3[:, :dj]], 1)
    return x3


def _conv3x3(x3, w_ref, b_ref, cout, offs=(-1, 0, 1)):
    s, _, c = x3.shape
    xb = x3.astype(jnp.bfloat16)
    xcat = jnp.concatenate([_shift2(xb, offs[0], 0), _shift2(xb, offs[1], 0),
                            _shift2(xb, offs[2], 0)], 2)
    yw = jnp.dot(xcat.reshape(s * s, 3 * c), w_ref[...],
                 preferred_element_type=jnp.float32)
    y3 = yw.reshape(s, s, 3 * cout)
    acc = b_ref[...].reshape(1, 1, cout)
    for kj in range(3):
        acc = acc + _shift2(y3[:, :, kj * cout:(kj + 1) * cout], 0, offs[kj])
    return acc.reshape(s * s, cout)


def _mask_sq(y, s, c, valid):
    ii = jax.lax.broadcasted_iota(jnp.int32, (s, s, 1), 0)
    jj = jax.lax.broadcasted_iota(jnp.int32, (s, s, 1), 1)
    return jnp.where((ii < valid) & (jj < valid), y.reshape(s, s, c), 0.0)


def _conv4x4s2(xref, w_ref, b_ref):
    s, _, cin = xref.shape
    half = s // 2
    par = [[xref[slice(p, s, 2), slice(q, s, 2), :] for q in range(2)]
           for p in range(2)]
    acc = b_ref[...]
    for ki in range(4):
        a_, p_ = (ki - 1) >> 1, (ki - 1) & 1
        for kj in range(4):
            b_, q_ = (kj - 1) >> 1, (kj - 1) & 1
            xs = _shift2(par[p_][q_], a_, b_).reshape(half * half, cin)
            acc = acc + jnp.dot(xs.astype(jnp.bfloat16), w_ref[ki * 4 + kj],
                                preferred_element_type=jnp.float32)
    return acc


def _head_kernel(rep1_ref, rep2_ref, ag1_ref, ag2_ref,
                 c1w_ref, c1b_ref, c2w_ref, c2b_ref, c3w_ref, c3b_ref,
                 d1w_ref, d1b_ref, d2w_ref, d2b_ref, d3w_ref, d3b_ref,
                 d4w_ref, d4b_ref, a1w_ref, a1b_ref, a2w_ref, a2b_ref,
                 res_ref, rm2_ref, s1_ref, s2_ref, s3_ref):
    def conv_stack(rep):
        y = jnp.maximum(_conv3x3(rep.reshape(H, W, GE), c1w_ref, c1b_ref, 256), 0.0)
        y = jnp.maximum(_conv3x3(y.reshape(H, W, 256), c2w_ref, c2b_ref, 128), 0.0)
        return jnp.tanh(_conv3x3(y.reshape(H, W, 128), c3w_ref, c3b_ref, GE))

    rm1 = conv_stack(rep1_ref[0])
    rm2 = conv_stack(rep2_ref[0])
    rm2_ref[0] = rm2

    xa = jnp.concatenate([rm1[0:1, :], ag1_ref[0], rm2[0:1, :], ag2_ref[0]],
                         axis=1)
    ha = jnp.maximum(jnp.dot(xa, a1w_ref[...],
                             preferred_element_type=jnp.float32) + a1b_ref[...],
                     0.0)
    ares = jnp.dot(ha, a2w_ref[...],
                   preferred_element_type=jnp.float32) + a2b_ref[...]

    xm = jnp.concatenate([rm1, rm2], axis=1).reshape(H, W, 2 * GE)
    y = jnp.maximum(_conv3x3(xm, d1w_ref, d1b_ref, 32, offs=(0, 1, 2)), 0.0)
    s1_ref[...] = _mask_sq(y, H, 32, 62)
    y = jnp.maximum(_conv4x4s2(s1_ref, d2w_ref, d2b_ref), 0.0)
    s2_ref[...] = _mask_sq(y, 32, 16, 31)
    y = jnp.maximum(_conv4x4s2(s2_ref, d3w_ref, d3b_ref), 0.0)
    s3_ref[...] = _mask_sq(y, 16, 8, 15)
    y = _conv4x4s2(s3_ref, d4w_ref, d4b_ref)
    y88 = y.reshape(8, 8, 1)[:, :, 0]
    res = (jax.nn.sigmoid(y88) + jax.nn.sigmoid(ares)) / 2.0
    res_ref[0, 0] = res[0:7, 0:7]


def _full(shape):
    return pl.BlockSpec(shape, lambda i: (0,) * len(shape))


def kernel(x1, src1, dst1, mp1, x2, src2, dst2, mp2,
           W1, W2, W3, c1w, c1b, c2w, c2b, c3w, c3b,
           d1w, d1b, d2w, d2b, d3w, d3b, d4w, d4b,
           a1w, a1b, a2w, a2b):
    f32 = jnp.float32
    offs = (jnp.arange(B, dtype=jnp.int32) * N)[:, None]
    xs = jnp.concatenate([x1.reshape(B, N, D), x2.reshape(B, N, D)], 0)
    srcs = jnp.concatenate([(src1.reshape(B, EPG) - offs),
                            (src2.reshape(B, EPG) - offs)], 0).reshape(2 * B, 1, EPG)
    dsts = jnp.concatenate([(dst1.reshape(B, EPG) - offs),
                            (dst2.reshape(B, EPG) - offs)], 0).reshape(2 * B, 1, EPG)
    mps = jnp.concatenate([mp1.reshape(B, 1, P), mp2.reshape(B, 1, P)], 0)

    rep, ag = pl.pallas_call(
        _graph_kernel,
        grid=(2 * B,),
        in_specs=[
            pl.BlockSpec((1, N, D), lambda i: (i, 0, 0)),
            pl.BlockSpec((1, 1, EPG), lambda i: (i, 0, 0)),
            pl.BlockSpec((1, 1, EPG), lambda i: (i, 0, 0)),
            pl.BlockSpec((1, 1, P), lambda i: (i, 0, 0)),
            pl.BlockSpec((D, D), lambda i: (0, 0)),
            pl.BlockSpec((D, D), lambda i: (0, 0)),
            pl.BlockSpec((D, GE), lambda i: (0, 0)),
        ],
        out_specs=[
            pl.BlockSpec((1, P, GE), lambda i: (i, 0, 0)),
            pl.BlockSpec((1, 1, GE), lambda i: (i, 0, 0)),
        ],
        out_shape=[
            jax.ShapeDtypeStruct((2 * B, P, GE), jnp.bfloat16),
            jax.ShapeDtypeStruct((2 * B, 1, GE), f32),
        ],
        compiler_params=pltpu.CompilerParams(
            dimension_semantics=("parallel",),
            vmem_limit_bytes=48 * 1024 * 1024,
        ),
        name="graph_conv_map",
    )(xs, srcs, dsts, mps, W1, W2, W3)

    def taps(w):
        return jnp.transpose(w, (2, 3, 1, 0)).reshape(-1, w.shape[1], w.shape[0])

    def btaps(w):
        return taps(w).astype(jnp.bfloat16)

    def vtaps(w):
        t = jnp.transpose(w, (2, 1, 3, 0))
        return t.reshape(3 * w.shape[1], 3 * w.shape[0]).astype(jnp.bfloat16)

    res, rm2 = pl.pallas_call(
        _head_kernel,
        grid=(B,),
        in_specs=[
            pl.BlockSpec((1, P, GE), lambda i: (i, 0, 0)),
            pl.BlockSpec((1, P, GE), lambda i: (i, 0, 0)),
            pl.BlockSpec((1, 1, GE), lambda i: (i, 0, 0)),
            pl.BlockSpec((1, 1, GE), lambda i: (i, 0, 0)),
            _full((3 * GE, 3 * 256)), _full((1, 256)),
            _full((3 * 256, 3 * 128)), _full((1, 128)),
            _full((3 * 128, 3 * GE)), _full((1, GE)),
            _full((3 * 128, 3 * 32)), _full((1, 32)),
            _full((16, 32, 16)), _full((1, 16)),
            _full((16, 16, 8)), _full((1, 8)),
            _full((16, 8, 1)), _full((1, 1)),
            _full((2 * (64 + GE), 32)), _full((1, 32)),
            _full((32, 1)), _full((1, 1)),
        ],
        out_specs=[
            pl.BlockSpec((1, 1, 7, 7), lambda i: (i, 0, 0, 0)),
            pl.BlockSpec((1, P, GE), lambda i: (i, 0, 0)),
        ],
        out_shape=[
            jax.ShapeDtypeStruct((B, 1, 7, 7), f32),
            jax.ShapeDtypeStruct((B, P, GE), f32),
        ],
        scratch_shapes=[
            pltpu.VMEM((H, W, 32), jnp.float32),
            pltpu.VMEM((32, 32, 16), jnp.float32),
            pltpu.VMEM((16, 16, 8), jnp.float32),
        ],
        compiler_params=pltpu.CompilerParams(
            dimension_semantics=("parallel",),
            vmem_limit_bytes=48 * 1024 * 1024,
        ),
        name="conv_head",
    )(rep[:B], rep[B:], ag[:B], ag[B:],
      vtaps(c1w), c1b[None, :], vtaps(c2w), c2b[None, :], vtaps(c3w), c3b[None, :],
      vtaps(d1w), d1b[None, :], btaps(d2w), d2b[None, :], btaps(d3w), d3b[None, :],
      btaps(d4w), d4b[None, :],
      a1w.T, a1b[None, :], a2w.T, a2b[None, :])

    rm2_nchw = jnp.transpose(rm2.reshape(B, H, W, GE), (0, 3, 1, 2))
    return res, rm2_nchw
